# baseline (speedup 1.0000x reference)
import jax
import jax.numpy as jnp
from jax.experimental import pallas as pl
from jax.experimental.pallas import tpu as pltpu

_THRES = 0.5
_L = 4096
_BR = 512
_NB = _L // _BR
_WC = 2048
_BIG = _L * _L


def _stats_chunk(x, k, ac_ref, ar_ref):
    col = jax.lax.broadcasted_iota(jnp.int32, (_BR, _L), 1)
    g = k * _BR + jax.lax.broadcasted_iota(jnp.int32, (_BR, 1), 0)
    masked = jnp.where(col == g, -jnp.inf, x)
    vmax = jnp.max(masked, axis=1, keepdims=True)
    cand = jnp.where(masked == vmax, col, _L)
    inds = jnp.min(cand, axis=1, keepdims=True)
    a = jnp.where(vmax > _THRES, inds, -1)
    ac_ref[pl.ds(k * _BR, _BR), :] = a
    krow = jax.lax.broadcasted_iota(jnp.int32, (_BR, _BR), 0)
    kcol = jax.lax.broadcasted_iota(jnp.int32, (_BR, _BR), 1)
    spread = jnp.where(krow == kcol, a, _BIG)
    ar_ref[0:1, pl.ds(k * _BR, _BR)] = jnp.min(spread, axis=0, keepdims=True)


def _block_value(bi, bh, ac_ref, ar_ref):
    rowi = jax.lax.broadcasted_iota(jnp.int32, (_BR, _WC), 0)
    coli = jax.lax.broadcasted_iota(jnp.int32, (_BR, _WC), 1)
    g = rowi + bi * _BR
    jg = coli + bh * _WC
    a_i = ac_ref[pl.ds(bi * _BR, _BR), :]
    a_j = ar_ref[0:1, pl.ds(bh * _WC, _WC)]
    hit = (jg == g) | (jg == a_i) | (a_j == g)
    return hit.astype(jnp.float32)


def _fused_body(x_hbm, out_hbm, xbuf, obuf, ac_ref, ar_ref, in_sem, out_sem):
    def in_copy(k, buf):
        return pltpu.make_async_copy(
            x_hbm.at[pl.ds(k * _BR, _BR), :], xbuf.at[buf], in_sem.at[buf]
        )

    def out_copy(bi, bh, buf):
        return pltpu.make_async_copy(
            obuf.at[buf],
            out_hbm.at[pl.ds(bi * _BR, _BR), pl.ds(bh * _WC, _WC)],
            out_sem.at[buf],
        )

    wsched = [(i, 0) for i in range(_NB)] + [(i, 1) for i in range(_NB)]

    in_copy(0, 0).start()
    nw = 0
    for k in range(_NB):
        if k + 1 < _NB:
            in_copy(k + 1, (k + 1) % 2).start()
        in_copy(k, k % 2).wait()
        _stats_chunk(xbuf[k % 2], k, ac_ref, ar_ref)
        if k >= 4:
            bi, bh = wsched[nw]
            buf = nw % 2
            if nw >= 2:
                out_copy(*wsched[nw - 2], buf).wait()
            obuf[buf] = _block_value(bi, bh, ac_ref, ar_ref)
            out_copy(bi, bh, buf).start()
            nw += 1
    while nw < len(wsched):
        bi, bh = wsched[nw]
        buf = nw % 2
        if nw >= 2:
            out_copy(*wsched[nw - 2], buf).wait()
        obuf[buf] = _block_value(bi, bh, ac_ref, ar_ref)
        out_copy(bi, bh, buf).start()
        nw += 1
    out_copy(*wsched[nw - 2], nw % 2).wait()
    out_copy(*wsched[nw - 1], (nw - 1) % 2).wait()


def kernel(input):
    x = input.reshape(_L, _L)

    out2d = pl.pallas_call(
        _fused_body,
        in_specs=[pl.BlockSpec(memory_space=pl.ANY)],
        out_specs=pl.BlockSpec(memory_space=pl.ANY),
        out_shape=jax.ShapeDtypeStruct((_L, _L), jnp.float32),
        scratch_shapes=[
            pltpu.VMEM((2, _BR, _L), jnp.float32),
            pltpu.VMEM((2, _BR, _WC), jnp.float32),
            pltpu.VMEM((_L, 1), jnp.int32),
            pltpu.VMEM((1, _L), jnp.int32),
            pltpu.SemaphoreType.DMA((2,)),
            pltpu.SemaphoreType.DMA((2,)),
        ],
    )(x)

    return out2d.reshape(input.shape)
